# baseline (speedup 1.0000x reference)
import jax
import jax.numpy as jnp
from jax import lax
from jax.experimental import pallas as pl
from jax.experimental.pallas import tpu as pltpu

_RATIO = 0.7
_H = 512
_W = 512
_C = 128
_BLK = 12


def _do_chunk(img_ref, out_ref, t, l, b, r, hc, wc):
    hlo, whi = hc * _C, wc * _C + _C
    hhi, wlo = hc * _C + _C, wc * _C
    hsl = slice(hlo, hhi)
    wsl = slice(wlo, whi)
    fully_in = (t <= hlo) & (hhi <= b) & (l <= wlo) & (whi <= r)
    fully_out = (hhi <= t) | (hlo >= b) | (whi <= l) | (wlo >= r)
    partial = jnp.logical_not(fully_in | fully_out)

    @pl.when(fully_in)
    def _():
        out_ref[:, hsl, wsl] = img_ref[:, hsl, wsl]

    @pl.when(fully_out)
    def _():
        out_ref[:, hsl, wsl] = jnp.full((_BLK, _C, _C), -1.0, jnp.float32)

    @pl.when(partial)
    def _():
        row_id = lax.broadcasted_iota(jnp.int32, (_C, _C), 0) + hlo
        col_id = lax.broadcasted_iota(jnp.int32, (_C, _C), 1) + wlo
        region = (row_id >= t) & (row_id < b) & (col_id >= l) & (col_id < r)
        out_ref[:, hsl, wsl] = jnp.where(
            region[None, :, :], img_ref[:, hsl, wsl], -1.0
        )


def _fused_body(mask_hbm, img_ref, out_ref, bbox, mvm, sem):
    @pl.when(pl.program_id(0) == 0)
    def _():
        cp = pltpu.make_async_copy(mask_hbm, mvm, sem)
        cp.start()
        cp.wait()
        m = mvm[...]
        nz = m != 0.0
        row_id = lax.broadcasted_iota(jnp.int32, (_H, _W), 0)
        col_id = lax.broadcasted_iota(jnp.int32, (_H, _W), 1)
        top = jnp.min(jnp.where(nz, row_id, _H))
        bottom = jnp.max(jnp.where(nz, row_id, -1))
        left = jnp.min(jnp.where(nz, col_id, _W))
        right = jnp.max(jnp.where(nz, col_id, -1))
        bbox[0] = jnp.floor(top * _RATIO).astype(jnp.int32)
        bbox[1] = jnp.floor(left * _RATIO).astype(jnp.int32)
        bbox[2] = jnp.floor(bottom + (_H - bottom) * (1.0 - _RATIO)).astype(jnp.int32)
        bbox[3] = jnp.floor(right + (_W - right) * (1.0 - _RATIO)).astype(jnp.int32)

    t = bbox[0]
    l = bbox[1]
    b = bbox[2]
    r = bbox[3]
    for hc in range(_H // _C):
        for wc in range(_W // _C):
            _do_chunk(img_ref, out_ref, t, l, b, r, hc, wc)


@jax.jit
def kernel(image, cover, mask):
    del cover
    m = mask[0, 0]
    n = image.shape[0] * image.shape[1]
    x = image.reshape(n, _H, _W)
    out = pl.pallas_call(
        _fused_body,
        grid=(n // _BLK,),
        in_specs=[
            pl.BlockSpec(memory_space=pl.ANY),
            pl.BlockSpec((_BLK, _H, _W), lambda i: (i, 0, 0)),
        ],
        out_specs=pl.BlockSpec((_BLK, _H, _W), lambda i: (i, 0, 0)),
        out_shape=jax.ShapeDtypeStruct((n, _H, _W), jnp.float32),
        scratch_shapes=[
            pltpu.SMEM((4,), jnp.int32),
            pltpu.VMEM((_H, _W), jnp.float32),
            pltpu.SemaphoreType.DMA,
        ],
        compiler_params=pltpu.CompilerParams(
            dimension_semantics=("arbitrary",),
        ),
    )(m, x)
    return out.reshape(image.shape)


# speedup vs baseline: 1.0164x; 1.0164x over previous
import jax
import jax.numpy as jnp
from jax import lax
from jax.experimental import pallas as pl
from jax.experimental.pallas import tpu as pltpu

_RATIO = 0.7
_H = 512
_W = 512
_BLK = 12
_RS = 8


def _bbox_body(mask_ref, bbox_ref):
    m = mask_ref[...]
    nz = m != 0.0
    row_id = lax.broadcasted_iota(jnp.int32, (_H, _W), 0)
    col_id = lax.broadcasted_iota(jnp.int32, (_H, _W), 1)
    top = jnp.min(jnp.where(nz, row_id, _H))
    bottom = jnp.max(jnp.where(nz, row_id, -1))
    left = jnp.min(jnp.where(nz, col_id, _W))
    right = jnp.max(jnp.where(nz, col_id, -1))
    bbox_ref[0] = jnp.floor(top * _RATIO).astype(jnp.int32)
    bbox_ref[1] = jnp.floor(left * _RATIO).astype(jnp.int32)
    bbox_ref[2] = jnp.floor(bottom + (_H - bottom) * (1.0 - _RATIO)).astype(jnp.int32)
    bbox_ref[3] = jnp.floor(right + (_W - right) * (1.0 - _RATIO)).astype(jnp.int32)


def _crop_body(bbox_ref, img_ref, out_ref):
    t = bbox_ref[0]
    l = bbox_ref[1]
    b = bbox_ref[2]
    r = bbox_ref[3]
    col_id = lax.broadcasted_iota(jnp.int32, (_RS, _W), 1)
    col_in = (col_id >= l) & (col_id < r)
    for s in range(_H // _RS):
        rs = s * _RS
        row_id = lax.broadcasted_iota(jnp.int32, (_RS, _W), 0) + rs
        reg = (row_id >= t) & (row_id < b) & col_in
        out_ref[:, rs : rs + _RS, :] = jnp.where(
            reg[None, :, :], img_ref[:, rs : rs + _RS, :], -1.0
        )


@jax.jit
def kernel(image, cover, mask):
    del cover
    m = mask[0, 0]
    bbox = pl.pallas_call(
        _bbox_body,
        out_shape=jax.ShapeDtypeStruct((4,), jnp.int32),
        in_specs=[pl.BlockSpec(memory_space=pltpu.VMEM)],
        out_specs=pl.BlockSpec(memory_space=pltpu.SMEM),
    )(m)

    n = image.shape[0] * image.shape[1]
    x = image.reshape(n, _H, _W)
    grid_spec = pltpu.PrefetchScalarGridSpec(
        num_scalar_prefetch=1,
        grid=(n // _BLK,),
        in_specs=[pl.BlockSpec((_BLK, _H, _W), lambda i, bbox: (i, 0, 0))],
        out_specs=pl.BlockSpec((_BLK, _H, _W), lambda i, bbox: (i, 0, 0)),
    )
    out = pl.pallas_call(
        _crop_body,
        grid_spec=grid_spec,
        out_shape=jax.ShapeDtypeStruct((n, _H, _W), jnp.float32),
    )(bbox, x)
    return out.reshape(image.shape)


# speedup vs baseline: 1.0170x; 1.0006x over previous
import jax
import jax.numpy as jnp
from jax import lax
from jax.experimental import pallas as pl
from jax.experimental.pallas import tpu as pltpu

_RATIO = 0.7
_H = 512
_W = 512
_BLK = 12
_N = 96 // _BLK
_RB = 504
_CB = 384


def _in_copies(img_hbm, bulk, bband, cband, sem_in, k, base):
    return [
        pltpu.make_async_copy(
            img_hbm.at[pl.ds(base, _BLK), pl.ds(0, _RB), pl.ds(0, _CB)],
            bulk.at[k],
            sem_in.at[k],
        ),
        pltpu.make_async_copy(
            img_hbm.at[pl.ds(base, _BLK), pl.ds(0, _RB), pl.ds(_CB, _W - _CB)],
            bband.at[k],
            sem_in.at[k],
        ),
        pltpu.make_async_copy(
            img_hbm.at[pl.ds(base, _BLK), pl.ds(_RB, _H - _RB), pl.ds(0, _W)],
            cband.at[k],
            sem_in.at[k],
        ),
    ]


def _out_copies(out_hbm, bulk, bband, cband, sem_out, k, base):
    return [
        pltpu.make_async_copy(
            bulk.at[k],
            out_hbm.at[pl.ds(base, _BLK), pl.ds(0, _RB), pl.ds(0, _CB)],
            sem_out.at[k],
        ),
        pltpu.make_async_copy(
            bband.at[k],
            out_hbm.at[pl.ds(base, _BLK), pl.ds(0, _RB), pl.ds(_CB, _W - _CB)],
            sem_out.at[k],
        ),
        pltpu.make_async_copy(
            cband.at[k],
            out_hbm.at[pl.ds(base, _BLK), pl.ds(_RB, _H - _RB), pl.ds(0, _W)],
            sem_out.at[k],
        ),
    ]


def _region(t, l, b, r, shape, roff, coff):
    row_id = lax.broadcasted_iota(jnp.int32, shape, 0) + roff
    col_id = lax.broadcasted_iota(jnp.int32, shape, 1) + coff
    return (row_id >= t) & (row_id < b) & (col_id >= l) & (col_id < r)


def _body(mask_hbm, img_hbm, out_hbm, bbox, mvm, bulk, bband, cband,
          sem_in, sem_out, msem):
    i = pl.program_id(0)

    @pl.when(i == 0)
    def _():
        cp = pltpu.make_async_copy(mask_hbm, mvm, msem)
        cp.start()
        cp.wait()
        m = mvm[...]
        nz = m != 0.0
        row_id = lax.broadcasted_iota(jnp.int32, (_H, _W), 0)
        col_id = lax.broadcasted_iota(jnp.int32, (_H, _W), 1)
        top = jnp.min(jnp.where(nz, row_id, _H))
        bottom = jnp.max(jnp.where(nz, row_id, -1))
        left = jnp.min(jnp.where(nz, col_id, _W))
        right = jnp.max(jnp.where(nz, col_id, -1))
        bbox[0] = jnp.floor(top * _RATIO).astype(jnp.int32)
        bbox[1] = jnp.floor(left * _RATIO).astype(jnp.int32)
        bbox[2] = jnp.floor(bottom + (_H - bottom) * (1.0 - _RATIO)).astype(jnp.int32)
        bbox[3] = jnp.floor(right + (_W - right) * (1.0 - _RATIO)).astype(jnp.int32)

    for k in (0, 1):
        @pl.when((i >= 2) & (i < _N) & (lax.rem(i, 2) == k))
        def _(k=k):
            for cp in _out_copies(out_hbm, bulk, bband, cband, sem_out, k,
                                  (i - 2) * _BLK):
                cp.wait()

        @pl.when((i < _N) & (lax.rem(i, 2) == k))
        def _(k=k):
            for cp in _in_copies(img_hbm, bulk, bband, cband, sem_in, k,
                                 i * _BLK):
                cp.start()

    t = bbox[0]
    l = bbox[1]
    b = bbox[2]
    r = bbox[3]
    for k in (0, 1):
        @pl.when((i >= 1) & (lax.rem(i - 1, 2) == k))
        def _(k=k):
            j = i - 1
            base = j * _BLK
            for cp in _in_copies(img_hbm, bulk, bband, cband, sem_in, k, base):
                cp.wait()
            regb = _region(t, l, b, r, (_RB, _W - _CB), 0, _CB)
            bband[k] = jnp.where(regb[None, :, :], bband[k], -1.0)
            regc = _region(t, l, b, r, (_H - _RB, _W), _RB, 0)
            cband[k] = jnp.where(regc[None, :, :], cband[k], -1.0)

            bulk_inside = (t <= 0) & (l <= 0) & (b >= _RB) & (r >= _CB)

            @pl.when(jnp.logical_not(bulk_inside))
            def _():
                regm = _region(t, l, b, r, (_RB, _CB), 0, 0)
                bulk[k] = jnp.where(regm[None, :, :], bulk[k], -1.0)

            for cp in _out_copies(out_hbm, bulk, bband, cband, sem_out, k, base):
                cp.start()

        jlast = _N - 1 if (_N - 1) % 2 == k else _N - 2

        @pl.when(i == _N)
        def _(k=k, jlast=jlast):
            for cp in _out_copies(out_hbm, bulk, bband, cband, sem_out, k,
                                  jlast * _BLK):
                cp.wait()


@jax.jit
def kernel(image, cover, mask):
    del cover
    m = mask[0, 0]
    n = image.shape[0] * image.shape[1]
    x = image.reshape(n, _H, _W)
    out = pl.pallas_call(
        _body,
        grid=(_N + 1,),
        in_specs=[
            pl.BlockSpec(memory_space=pl.ANY),
            pl.BlockSpec(memory_space=pl.ANY),
        ],
        out_specs=pl.BlockSpec(memory_space=pl.ANY),
        out_shape=jax.ShapeDtypeStruct((n, _H, _W), jnp.float32),
        scratch_shapes=[
            pltpu.SMEM((4,), jnp.int32),
            pltpu.VMEM((_H, _W), jnp.float32),
            pltpu.VMEM((2, _BLK, _RB, _CB), jnp.float32),
            pltpu.VMEM((2, _BLK, _RB, _W - _CB), jnp.float32),
            pltpu.VMEM((2, _BLK, _H - _RB, _W), jnp.float32),
            pltpu.SemaphoreType.DMA((2,)),
            pltpu.SemaphoreType.DMA((2,)),
            pltpu.SemaphoreType.DMA,
        ],
        compiler_params=pltpu.CompilerParams(
            dimension_semantics=("arbitrary",),
        ),
    )(m, x)
    return out.reshape(image.shape)


# speedup vs baseline: 1.0222x; 1.0051x over previous
import jax
import jax.numpy as jnp
from jax import lax
from jax.experimental import pallas as pl
from jax.experimental.pallas import tpu as pltpu

_RATIO = 0.7
_H = 512
_W = 512
_BLK = 12
_N = 96 // _BLK
_RS = 504
_CS = 384


def _in_copy(img_hbm, buf, sem_in, k, base):
    return pltpu.make_async_copy(
        img_hbm.at[pl.ds(base, _BLK)], buf.at[k], sem_in.at[k]
    )


def _out_copy(out_hbm, buf, sem_out, k, base):
    return pltpu.make_async_copy(
        buf.at[k], out_hbm.at[pl.ds(base, _BLK)], sem_out.at[k]
    )


def _region(t, l, b, r, shape, roff, coff):
    row_id = lax.broadcasted_iota(jnp.int32, shape, 0) + roff
    col_id = lax.broadcasted_iota(jnp.int32, shape, 1) + coff
    return (row_id >= t) & (row_id < b) & (col_id >= l) & (col_id < r)


def _body(mask_hbm, img_hbm, out_hbm, bbox, mvm, buf, sem_in, sem_out, msem):
    i = pl.program_id(0)

    @pl.when(i == 0)
    def _():
        cp = pltpu.make_async_copy(mask_hbm, mvm, msem)
        cp.start()
        cp.wait()
        m = mvm[...]
        nz = m != 0.0
        row_id = lax.broadcasted_iota(jnp.int32, (_H, _W), 0)
        col_id = lax.broadcasted_iota(jnp.int32, (_H, _W), 1)
        top = jnp.min(jnp.where(nz, row_id, _H))
        bottom = jnp.max(jnp.where(nz, row_id, -1))
        left = jnp.min(jnp.where(nz, col_id, _W))
        right = jnp.max(jnp.where(nz, col_id, -1))
        bbox[0] = jnp.floor(top * _RATIO).astype(jnp.int32)
        bbox[1] = jnp.floor(left * _RATIO).astype(jnp.int32)
        bbox[2] = jnp.floor(bottom + (_H - bottom) * (1.0 - _RATIO)).astype(jnp.int32)
        bbox[3] = jnp.floor(right + (_W - right) * (1.0 - _RATIO)).astype(jnp.int32)

    for k in (0, 1):
        @pl.when((i >= 2) & (i < _N) & (lax.rem(i, 2) == k))
        def _(k=k):
            _out_copy(out_hbm, buf, sem_out, k, (i - 2) * _BLK).wait()

        @pl.when((i < _N) & (lax.rem(i, 2) == k))
        def _(k=k):
            _in_copy(img_hbm, buf, sem_in, k, i * _BLK).start()

    t = bbox[0]
    l = bbox[1]
    b = bbox[2]
    r = bbox[3]
    for k in (0, 1):
        @pl.when((i >= 1) & (lax.rem(i - 1, 2) == k))
        def _(k=k):
            base = (i - 1) * _BLK
            _in_copy(img_hbm, buf, sem_in, k, base).wait()

            strips_ok = (t <= 0) & (l <= 0) & (b >= _RS) & (r >= _CS)

            @pl.when(strips_ok)
            def _():
                regc = _region(t, l, b, r, (_H, _W - _CS), 0, _CS)
                buf[k, :, :, _CS:] = jnp.where(
                    regc[None, :, :], buf[k, :, :, _CS:], -1.0
                )
                regr = _region(t, l, b, r, (_H - _RS, _CS), _RS, 0)
                buf[k, :, _RS:, :_CS] = jnp.where(
                    regr[None, :, :], buf[k, :, _RS:, :_CS], -1.0
                )

            @pl.when(jnp.logical_not(strips_ok))
            def _():
                regf = _region(t, l, b, r, (_H, _W), 0, 0)
                buf[k] = jnp.where(regf[None, :, :], buf[k], -1.0)

            _out_copy(out_hbm, buf, sem_out, k, base).start()

        jlast = _N - 1 if (_N - 1) % 2 == k else _N - 2

        @pl.when(i == _N)
        def _(k=k, jlast=jlast):
            _out_copy(out_hbm, buf, sem_out, k, jlast * _BLK).wait()


@jax.jit
def kernel(image, cover, mask):
    del cover
    m = mask[0, 0]
    n = image.shape[0] * image.shape[1]
    x = image.reshape(n, _H, _W)
    out = pl.pallas_call(
        _body,
        grid=(_N + 1,),
        in_specs=[
            pl.BlockSpec(memory_space=pl.ANY),
            pl.BlockSpec(memory_space=pl.ANY),
        ],
        out_specs=pl.BlockSpec(memory_space=pl.ANY),
        out_shape=jax.ShapeDtypeStruct((n, _H, _W), jnp.float32),
        scratch_shapes=[
            pltpu.SMEM((4,), jnp.int32),
            pltpu.VMEM((_H, _W), jnp.float32),
            pltpu.VMEM((2, _BLK, _H, _W), jnp.float32),
            pltpu.SemaphoreType.DMA((2,)),
            pltpu.SemaphoreType.DMA((2,)),
            pltpu.SemaphoreType.DMA,
        ],
        compiler_params=pltpu.CompilerParams(
            dimension_semantics=("arbitrary",),
        ),
    )(m, x)
    return out.reshape(image.shape)


# speedup vs baseline: 1.0244x; 1.0022x over previous
import jax
import jax.numpy as jnp
from jax import lax
from jax.experimental import pallas as pl
from jax.experimental.pallas import tpu as pltpu

_RATIO = 0.7
_H = 512
_W = 512
_BLK = 12
_N = 96 // _BLK
_P = 3
_RS = 504
_CS = 384


def _in_copy(img_hbm, buf, sem_in, k, base):
    return pltpu.make_async_copy(
        img_hbm.at[pl.ds(base, _BLK)], buf.at[k], sem_in.at[k]
    )


def _out_copy(out_hbm, buf, sem_out, k, base):
    return pltpu.make_async_copy(
        buf.at[k], out_hbm.at[pl.ds(base, _BLK)], sem_out.at[k]
    )


def _region(t, l, b, r, shape, roff, coff):
    row_id = lax.broadcasted_iota(jnp.int32, shape, 0) + roff
    col_id = lax.broadcasted_iota(jnp.int32, shape, 1) + coff
    return (row_id >= t) & (row_id < b) & (col_id >= l) & (col_id < r)


def _body(mask_hbm, img_hbm, out_hbm, bbox, mvm, buf, sem_in, sem_out, msem):
    i = pl.program_id(0)

    @pl.when(i == 0)
    def _():
        cp = pltpu.make_async_copy(mask_hbm, mvm, msem)
        cp.start()
        cp.wait()
        m = mvm[...]
        nz = m != 0.0
        row_id = lax.broadcasted_iota(jnp.int32, (_H, _W), 0)
        col_id = lax.broadcasted_iota(jnp.int32, (_H, _W), 1)
        top = jnp.min(jnp.where(nz, row_id, _H))
        bottom = jnp.max(jnp.where(nz, row_id, -1))
        left = jnp.min(jnp.where(nz, col_id, _W))
        right = jnp.max(jnp.where(nz, col_id, -1))
        bbox[0] = jnp.floor(top * _RATIO).astype(jnp.int32)
        bbox[1] = jnp.floor(left * _RATIO).astype(jnp.int32)
        bbox[2] = jnp.floor(bottom + (_H - bottom) * (1.0 - _RATIO)).astype(jnp.int32)
        bbox[3] = jnp.floor(right + (_W - right) * (1.0 - _RATIO)).astype(jnp.int32)

    for k in range(_P):
        @pl.when((i >= _P) & (i < _N) & (lax.rem(i, _P) == k))
        def _(k=k):
            _out_copy(out_hbm, buf, sem_out, k, (i - _P) * _BLK).wait()

        @pl.when((i < _N) & (lax.rem(i, _P) == k))
        def _(k=k):
            _in_copy(img_hbm, buf, sem_in, k, i * _BLK).start()

    t = bbox[0]
    l = bbox[1]
    b = bbox[2]
    r = bbox[3]
    for k in range(_P):
        @pl.when((i >= 1) & (lax.rem(i - 1, _P) == k))
        def _(k=k):
            base = (i - 1) * _BLK
            _in_copy(img_hbm, buf, sem_in, k, base).wait()

            strips_ok = (t <= 0) & (l <= 0) & (b >= _RS) & (r >= _CS)

            @pl.when(strips_ok)
            def _():
                regc = _region(t, l, b, r, (_H, _W - _CS), 0, _CS)
                buf[k, :, :, _CS:] = jnp.where(
                    regc[None, :, :], buf[k, :, :, _CS:], -1.0
                )
                regr = _region(t, l, b, r, (_H - _RS, _CS), _RS, 0)
                buf[k, :, _RS:, :_CS] = jnp.where(
                    regr[None, :, :], buf[k, :, _RS:, :_CS], -1.0
                )

            @pl.when(jnp.logical_not(strips_ok))
            def _():
                regf = _region(t, l, b, r, (_H, _W), 0, 0)
                buf[k] = jnp.where(regf[None, :, :], buf[k], -1.0)

            _out_copy(out_hbm, buf, sem_out, k, base).start()

        jlast = max(j for j in range(_N) if j % _P == k and j + _P >= _N)

        @pl.when(i == _N)
        def _(k=k, jlast=jlast):
            _out_copy(out_hbm, buf, sem_out, k, jlast * _BLK).wait()


@jax.jit
def kernel(image, cover, mask):
    del cover
    m = mask[0, 0]
    n = image.shape[0] * image.shape[1]
    x = image.reshape(n, _H, _W)
    out = pl.pallas_call(
        _body,
        grid=(_N + 1,),
        in_specs=[
            pl.BlockSpec(memory_space=pl.ANY),
            pl.BlockSpec(memory_space=pl.ANY),
        ],
        out_specs=pl.BlockSpec(memory_space=pl.ANY),
        out_shape=jax.ShapeDtypeStruct((n, _H, _W), jnp.float32),
        scratch_shapes=[
            pltpu.SMEM((4,), jnp.int32),
            pltpu.VMEM((_H, _W), jnp.float32),
            pltpu.VMEM((_P, _BLK, _H, _W), jnp.float32),
            pltpu.SemaphoreType.DMA((_P,)),
            pltpu.SemaphoreType.DMA((_P,)),
            pltpu.SemaphoreType.DMA,
        ],
        compiler_params=pltpu.CompilerParams(
            dimension_semantics=("arbitrary",),
        ),
    )(m, x)
    return out.reshape(image.shape)
